# baseline (speedup 1.0000x reference)
import functools

import jax
import jax.numpy as jnp
from jax import lax
from jax.experimental import pallas as pl
from jax.experimental.pallas import tpu as pltpu
from jax.experimental.pallas import tpu_sc as plsc

_BATCH = 16384
_HIST = 50
_DIM = 32
_NCLASS = 1000

_NC = 2
_NS = 16
_NW = _NC * _NS
_LANES = 16
_BAGS_PER_W = _BATCH // _NW
_CHUNK_BAGS = 2
_IDX_PER_CHUNK = _CHUNK_BAGS * _HIST
_NCHUNK = _BAGS_PER_W // _CHUNK_BAGS

_sc_mesh = plsc.VectorSubcoreMesh(core_axis_name="c", subcore_axis_name="s")


def _tree_add(vals):
    while len(vals) > 1:
        nxt = [a + b for a, b in zip(vals[0::2], vals[1::2])]
        if len(vals) % 2:
            nxt.append(vals[-1])
        vals = nxt
    return vals[0]


@functools.partial(
    pl.kernel,
    out_type=jax.ShapeDtypeStruct((_BATCH, _DIM), jnp.float32),
    mesh=_sc_mesh,
    scratch_types=[
        pltpu.VMEM((_NCHUNK, _IDX_PER_CHUNK), jnp.int32),
        pltpu.VMEM((_IDX_PER_CHUNK, _DIM), jnp.float32),
        pltpu.VMEM((_IDX_PER_CHUNK, _DIM), jnp.float32),
        pltpu.VMEM((_BAGS_PER_W, _DIM), jnp.float32),
        pltpu.SemaphoreType.DMA,
        pltpu.SemaphoreType.DMA,
    ],
    compiler_params=pltpu.CompilerParams(use_tc_tiling_on_sc=False),
)
def _bag_sum_sc(x_hbm, table_hbm, out_hbm, idx_v, g0, g1, acc_v, sem0, sem1):
    wid = lax.axis_index("c") * _NS + lax.axis_index("s")
    pltpu.sync_copy(x_hbm.at[wid], idx_v)
    pltpu.async_copy(table_hbm.at[idx_v.at[0]], g0, sem0)

    def reduce_chunk(g, chunk):
        for k in range(_CHUNK_BAGS):
            row = chunk * _CHUNK_BAGS + k
            for h in range(_DIM // _LANES):
                vals = [
                    g[k * _HIST + r, pl.ds(h * _LANES, _LANES)]
                    for r in range(_HIST)
                ]
                acc_v[row, pl.ds(h * _LANES, _LANES)] = _tree_add(vals)

    @pl.loop(0, _NCHUNK, step=2)
    def _(j):
        pltpu.async_copy(table_hbm.at[idx_v.at[j + 1]], g1, sem1)
        pltpu.make_async_copy(table_hbm.at[idx_v.at[j]], g0, sem0).wait()
        reduce_chunk(g0, j)

        @pl.when(j + 2 < _NCHUNK)
        def _():
            pltpu.async_copy(table_hbm.at[idx_v.at[j + 2]], g0, sem0)

        pltpu.make_async_copy(table_hbm.at[idx_v.at[j + 1]], g1, sem1).wait()
        reduce_chunk(g1, j + 1)

    pltpu.sync_copy(acc_v, out_hbm.at[pl.ds(wid * _BAGS_PER_W, _BAGS_PER_W)])


_TCW = 1024


def _tp_body(i_ref, o_ref):
    t = i_ref[...]
    tt = jnp.transpose(t)
    o_ref[...] = jnp.concatenate(
        [tt, jnp.zeros((_TCW, 128 - _DIM), jnp.float32)], axis=1
    )


def _table_padrows_tc(tT):
    n = tT.shape[1]
    grid = (n + _TCW - 1) // _TCW
    return pl.pallas_call(
        _tp_body,
        grid=(grid,),
        in_specs=[pl.BlockSpec((_DIM, _TCW), lambda i: (0, i))],
        out_specs=pl.BlockSpec((_TCW, 128), lambda i: (i, 0)),
        out_shape=jax.ShapeDtypeStruct((n, 128), jnp.float32),
    )(tT)


_BT = 512


def _mm_body(w_ref, e_ref, b_ref, o_ref):
    o_ref[...] = (
        lax.dot_general(
            w_ref[...],
            e_ref[...],
            (((1,), (1,)), ((), ())),
            preferred_element_type=jnp.float32,
            precision=lax.Precision.HIGHEST,
        )
        + b_ref[...]
    )


def _linear_tc(embed_sum, wt, bcol):
    return pl.pallas_call(
        _mm_body,
        grid=(_BATCH // _BT,),
        in_specs=[
            pl.BlockSpec((_NCLASS, _DIM), lambda i: (0, 0)),
            pl.BlockSpec((_BT, _DIM), lambda i: (i, 0)),
            pl.BlockSpec((_NCLASS, 1), lambda i: (0, 0)),
        ],
        out_specs=pl.BlockSpec((_NCLASS, _BT), lambda i: (0, i)),
        out_shape=jax.ShapeDtypeStruct((_NCLASS, _BATCH), jnp.float32),
    )(wt, embed_sum, bcol)


def kernel(x, table, W, b):
    xw = (x.astype(jnp.int32) * 4).reshape(_NW, _NCHUNK, _IDX_PER_CHUNK)
    n_rows = table.shape[0]
    tpad = _table_padrows_tc(jnp.transpose(table))
    tbl = jnp.reshape(tpad, (4 * n_rows, _DIM))
    embed_sum = _bag_sum_sc(xw, tbl)
    wt = W.astype(jnp.float32) * (1.0 / _HIST)
    bcol = b.reshape(_NCLASS, 1).astype(jnp.float32)
    out_t = _linear_tc(embed_sum, wt, bcol)
    return jnp.transpose(out_t)


# speedup vs baseline: 1.7606x; 1.7606x over previous
import functools

import jax
import jax.numpy as jnp
from jax import lax
from jax.experimental import pallas as pl
from jax.experimental.pallas import tpu as pltpu
from jax.experimental.pallas import tpu_sc as plsc

_BATCH = 16384
_HIST = 50
_DIM = 32
_NCLASS = 1000

_NC = 2
_NS = 16
_NW = _NC * _NS
_LANES = 16
_BAGS_PER_W = _BATCH // _NW
_CHUNK_BAGS = 2
_IDX_PER_CHUNK = _CHUNK_BAGS * _HIST
_NCHUNK = _BAGS_PER_W // _CHUNK_BAGS

_sc_mesh = plsc.VectorSubcoreMesh(core_axis_name="c", subcore_axis_name="s")


def _tree_add(vals):
    while len(vals) > 1:
        nxt = [a + b for a, b in zip(vals[0::2], vals[1::2])]
        if len(vals) % 2:
            nxt.append(vals[-1])
        vals = nxt
    return vals[0]


@functools.partial(
    pl.kernel,
    out_type=jax.ShapeDtypeStruct((_BATCH, _DIM), jnp.float32),
    mesh=_sc_mesh,
    scratch_types=[
        pltpu.VMEM((_NCHUNK, _IDX_PER_CHUNK), jnp.int32),
        pltpu.VMEM((_IDX_PER_CHUNK, _DIM), jnp.float32),
        pltpu.VMEM((_IDX_PER_CHUNK, _DIM), jnp.float32),
        pltpu.VMEM((_BAGS_PER_W, _DIM), jnp.float32),
        pltpu.SemaphoreType.DMA,
        pltpu.SemaphoreType.DMA,
    ],
    compiler_params=pltpu.CompilerParams(use_tc_tiling_on_sc=False),
)
def _bag_sum_sc(x_hbm, table_hbm, out_hbm, idx_v, g0, g1, acc_v, sem0, sem1):
    wid = lax.axis_index("c") * _NS + lax.axis_index("s")
    pltpu.sync_copy(x_hbm.at[wid], idx_v)
    pltpu.async_copy(table_hbm.at[idx_v.at[0]], g0, sem0)

    def reduce_chunk(g, chunk):
        for k in range(_CHUNK_BAGS):
            row = chunk * _CHUNK_BAGS + k
            for h in range(_DIM // _LANES):
                vals = [
                    g[k * _HIST + r, pl.ds(h * _LANES, _LANES)]
                    for r in range(_HIST)
                ]
                acc_v[row, pl.ds(h * _LANES, _LANES)] = _tree_add(vals)

    @pl.loop(0, _NCHUNK, step=2)
    def _(j):
        pltpu.async_copy(table_hbm.at[idx_v.at[j + 1]], g1, sem1)
        pltpu.make_async_copy(table_hbm.at[idx_v.at[j]], g0, sem0).wait()
        reduce_chunk(g0, j)

        @pl.when(j + 2 < _NCHUNK)
        def _():
            pltpu.async_copy(table_hbm.at[idx_v.at[j + 2]], g0, sem0)

        pltpu.make_async_copy(table_hbm.at[idx_v.at[j + 1]], g1, sem1).wait()
        reduce_chunk(g1, j + 1)

    pltpu.sync_copy(acc_v, out_hbm.at[pl.ds(wid * _BAGS_PER_W, _BAGS_PER_W)])


_TCW = 1024


_TPW = 2048
_TPB = 4 * _TPW
_TPMAIN = 122
_TPTAIL = 512
_TPROWS = (_TPMAIN + 1) * _TPW


def _tp_body(m0, m1, m2, m3, t0, t1, o_ref):
    pid = pl.program_id(0)
    main = jnp.concatenate(
        [jnp.transpose(r[...]) for r in (m0, m1, m2, m3)], axis=1
    )
    tail = jnp.concatenate(
        [
            jnp.transpose(t0[...]),
            jnp.transpose(t1[...]),
            jnp.zeros((_TPTAIL, 2 * _DIM), jnp.float32),
        ],
        axis=1,
    )
    tail = jnp.concatenate(
        [tail, jnp.zeros((_TPW - _TPTAIL, 128), jnp.float32)], axis=0
    )
    o_ref[...] = jnp.where(pid < _TPMAIN, main, tail)


def _table_rowgroups_tc(tT):
    max_blk = (tT.shape[1] - _TPW) // _TPW
    in_specs = [
        pl.BlockSpec(
            (_DIM, _TPW), lambda i, k=k: (0, jnp.minimum(4 * i + k, max_blk))
        )
        for k in range(4)
    ]
    in_specs += [
        pl.BlockSpec((_DIM, _TPTAIL), lambda i, k=k: (0, 1952 + k))
        for k in range(2)
    ]
    return pl.pallas_call(
        _tp_body,
        grid=(_TPMAIN + 1,),
        in_specs=in_specs,
        out_specs=pl.BlockSpec((_TPW, 128), lambda i: (i, 0)),
        out_shape=jax.ShapeDtypeStruct((_TPROWS, 128), jnp.float32),
    )(tT, tT, tT, tT, tT, tT)


_BT = 512


def _mm_body(w_ref, e_ref, b_ref, o_ref):
    o_ref[...] = (
        lax.dot_general(
            w_ref[...],
            e_ref[...],
            (((1,), (1,)), ((), ())),
            preferred_element_type=jnp.float32,
            precision=lax.Precision.HIGHEST,
        )
        + b_ref[...]
    )


def _linear_tc(embed_sum, wt, bcol):
    return pl.pallas_call(
        _mm_body,
        grid=(_BATCH // _BT,),
        in_specs=[
            pl.BlockSpec((_NCLASS, _DIM), lambda i: (0, 0)),
            pl.BlockSpec((_BT, _DIM), lambda i: (i, 0)),
            pl.BlockSpec((_NCLASS, 1), lambda i: (0, 0)),
        ],
        out_specs=pl.BlockSpec((_NCLASS, _BT), lambda i: (0, i)),
        out_shape=jax.ShapeDtypeStruct((_NCLASS, _BATCH), jnp.float32),
    )(wt, embed_sum, bcol)


def kernel(x, table, W, b):
    xi = x.astype(jnp.int32)
    b8 = xi % _TPB
    j_main = (xi - b8) + 4 * (b8 % _TPW) + b8 // _TPW
    t = xi - _TPMAIN * _TPB
    j_tail = _TPMAIN * _TPB + 4 * (t % _TPTAIL) + t // _TPTAIL
    xw = jnp.where(xi < _TPMAIN * _TPB, j_main, j_tail).reshape(
        _NW, _NCHUNK, _IDX_PER_CHUNK
    )
    tpack = _table_rowgroups_tc(jnp.transpose(table))
    tbl = jnp.reshape(tpack, (4 * _TPROWS, _DIM))
    embed_sum = _bag_sum_sc(xw, tbl)
    wt = W.astype(jnp.float32) * (1.0 / _HIST)
    bcol = b.reshape(_NCLASS, 1).astype(jnp.float32)
    out_t = _linear_tc(embed_sum, wt, bcol)
    return jnp.transpose(out_t)


# speedup vs baseline: 1.9444x; 1.1044x over previous
import functools

import jax
import jax.numpy as jnp
from jax import lax
from jax.experimental import pallas as pl
from jax.experimental.pallas import tpu as pltpu
from jax.experimental.pallas import tpu_sc as plsc

_BATCH = 16384
_HIST = 50
_DIM = 32
_NCLASS = 1000

_NC = 2
_NS = 16
_NW = _NC * _NS
_LANES = 16
_BAGS_PER_W = _BATCH // _NW
_CHUNK_BAGS = 2
_IDX_PER_CHUNK = _CHUNK_BAGS * _HIST
_NCHUNK = _BAGS_PER_W // _CHUNK_BAGS

_sc_mesh = plsc.VectorSubcoreMesh(core_axis_name="c", subcore_axis_name="s")


def _tree_add(vals):
    while len(vals) > 1:
        nxt = [a + b for a, b in zip(vals[0::2], vals[1::2])]
        if len(vals) % 2:
            nxt.append(vals[-1])
        vals = nxt
    return vals[0]


@functools.partial(
    pl.kernel,
    out_type=jax.ShapeDtypeStruct((_BATCH, _DIM), jnp.float32),
    mesh=_sc_mesh,
    scratch_types=[
        pltpu.VMEM((_NCHUNK, _IDX_PER_CHUNK), jnp.int32),
        pltpu.VMEM((_IDX_PER_CHUNK, _DIM), jnp.float32),
        pltpu.VMEM((_IDX_PER_CHUNK, _DIM), jnp.float32),
        pltpu.VMEM((_BAGS_PER_W, _DIM), jnp.float32),
        pltpu.SemaphoreType.DMA,
        pltpu.SemaphoreType.DMA,
    ],
    compiler_params=pltpu.CompilerParams(use_tc_tiling_on_sc=False),
)
def _bag_sum_sc(x_hbm, table_hbm, out_hbm, idx_v, g0, g1, acc_v, sem0, sem1):
    wid = lax.axis_index("c") * _NS + lax.axis_index("s")
    pltpu.sync_copy(x_hbm.at[wid], idx_v)
    pltpu.async_copy(table_hbm.at[idx_v.at[0]], g0, sem0)

    def reduce_chunk(g, chunk):
        for k in range(_CHUNK_BAGS):
            row = chunk * _CHUNK_BAGS + k
            for h in range(_DIM // _LANES):
                vals = [
                    g[k * _HIST + r, pl.ds(h * _LANES, _LANES)]
                    for r in range(_HIST)
                ]
                acc_v[row, pl.ds(h * _LANES, _LANES)] = _tree_add(vals)

    @pl.loop(0, _NCHUNK, step=2)
    def _(j):
        pltpu.async_copy(table_hbm.at[idx_v.at[j + 1]], g1, sem1)
        pltpu.make_async_copy(table_hbm.at[idx_v.at[j]], g0, sem0).wait()
        reduce_chunk(g0, j)

        @pl.when(j + 2 < _NCHUNK)
        def _():
            pltpu.async_copy(table_hbm.at[idx_v.at[j + 2]], g0, sem0)

        pltpu.make_async_copy(table_hbm.at[idx_v.at[j + 1]], g1, sem1).wait()
        reduce_chunk(g1, j + 1)

    pltpu.sync_copy(acc_v, out_hbm.at[pl.ds(wid * _BAGS_PER_W, _BAGS_PER_W)])


_TCW = 1024


_TPW = 2048
_TPB = 4 * _TPW
_TPMAIN = 122
_TPTAIL = 512
_TPROWS = (_TPMAIN + 1) * _TPW


def _tp_body(m0, m1, m2, m3, t0, t1, o_ref):
    pid = pl.program_id(0)

    @pl.when(pid < _TPMAIN)
    def _():
        o_ref[...] = jnp.concatenate(
            [jnp.transpose(r[...]) for r in (m0, m1, m2, m3)], axis=1
        )

    @pl.when(pid == _TPMAIN)
    def _():
        tail = jnp.concatenate(
            [
                jnp.transpose(t0[...]),
                jnp.transpose(t1[...]),
                jnp.zeros((_TPTAIL, 2 * _DIM), jnp.float32),
            ],
            axis=1,
        )
        o_ref[...] = jnp.concatenate(
            [tail, jnp.zeros((_TPW - _TPTAIL, 128), jnp.float32)], axis=0
        )


def _table_rowgroups_tc(tT):
    max_blk = (tT.shape[1] - _TPW) // _TPW
    in_specs = [
        pl.BlockSpec(
            (_DIM, _TPW), lambda i, k=k: (0, jnp.minimum(4 * i + k, max_blk))
        )
        for k in range(4)
    ]
    in_specs += [
        pl.BlockSpec((_DIM, _TPTAIL), lambda i, k=k: (0, 1952 + k))
        for k in range(2)
    ]
    return pl.pallas_call(
        _tp_body,
        grid=(_TPMAIN + 1,),
        in_specs=in_specs,
        out_specs=pl.BlockSpec((_TPW, 128), lambda i: (i, 0)),
        out_shape=jax.ShapeDtypeStruct((_TPROWS, 128), jnp.float32),
    )(tT, tT, tT, tT, tT, tT)


_BT = 512


def _mm_body(w_ref, e_ref, b_ref, o_ref):
    o_ref[...] = (
        lax.dot_general(
            w_ref[...],
            e_ref[...],
            (((1,), (1,)), ((), ())),
            preferred_element_type=jnp.float32,
        )
        + b_ref[...]
    )


def _linear_tc(embed_sum, wt, bcol):
    return pl.pallas_call(
        _mm_body,
        grid=(_BATCH // _BT,),
        in_specs=[
            pl.BlockSpec((_NCLASS, _DIM), lambda i: (0, 0)),
            pl.BlockSpec((_BT, _DIM), lambda i: (i, 0)),
            pl.BlockSpec((_NCLASS, 1), lambda i: (0, 0)),
        ],
        out_specs=pl.BlockSpec((_NCLASS, _BT), lambda i: (0, i)),
        out_shape=jax.ShapeDtypeStruct((_NCLASS, _BATCH), jnp.float32),
    )(wt, embed_sum, bcol)


def kernel(x, table, W, b):
    xi = x.astype(jnp.int32)
    b8 = xi % _TPB
    j_main = (xi - b8) + 4 * (b8 % _TPW) + b8 // _TPW
    t = xi - _TPMAIN * _TPB
    j_tail = _TPMAIN * _TPB + 4 * (t % _TPTAIL) + t // _TPTAIL
    xw = jnp.where(xi < _TPMAIN * _TPB, j_main, j_tail).reshape(
        _NW, _NCHUNK, _IDX_PER_CHUNK
    )
    tpack = _table_rowgroups_tc(jnp.transpose(table))
    tbl = jnp.reshape(tpack, (4 * _TPROWS, _DIM))
    embed_sum = _bag_sum_sc(xw, tbl)
    wt = W.astype(jnp.float32) * (1.0 / _HIST)
    bcol = b.reshape(_NCLASS, 1).astype(jnp.float32)
    out_t = _linear_tc(embed_sum, wt, bcol)
    return jnp.transpose(out_t)


# speedup vs baseline: 2.1966x; 1.1297x over previous
import functools

import jax
import jax.numpy as jnp
from jax import lax
from jax.experimental import pallas as pl
from jax.experimental.pallas import tpu as pltpu
from jax.experimental.pallas import tpu_sc as plsc

_BATCH = 16384
_HIST = 50
_DIM = 32
_NCLASS = 1000

_NC = 2
_NS = 16
_NW = _NC * _NS
_LANES = 16
_BAGS_PER_W = _BATCH // _NW
_CHUNK_BAGS = 2
_IDX_PER_CHUNK = _CHUNK_BAGS * _HIST
_NCHUNK = _BAGS_PER_W // _CHUNK_BAGS

_sc_mesh = plsc.VectorSubcoreMesh(core_axis_name="c", subcore_axis_name="s")


def _tree_add(vals):
    while len(vals) > 1:
        nxt = [a + b for a, b in zip(vals[0::2], vals[1::2])]
        if len(vals) % 2:
            nxt.append(vals[-1])
        vals = nxt
    return vals[0]


@functools.partial(
    pl.kernel,
    out_type=jax.ShapeDtypeStruct((_BATCH, _DIM), jnp.float32),
    mesh=_sc_mesh,
    scratch_types=[
        pltpu.VMEM((_NCHUNK, _IDX_PER_CHUNK), jnp.int32),
        pltpu.VMEM((_IDX_PER_CHUNK, _DIM), jnp.float32),
        pltpu.VMEM((_IDX_PER_CHUNK, _DIM), jnp.float32),
        pltpu.VMEM((_IDX_PER_CHUNK, _DIM), jnp.float32),
        pltpu.VMEM((_IDX_PER_CHUNK, _DIM), jnp.float32),
        pltpu.VMEM((_BAGS_PER_W, _DIM), jnp.float32),
        pltpu.SemaphoreType.DMA,
        pltpu.SemaphoreType.DMA,
        pltpu.SemaphoreType.DMA,
        pltpu.SemaphoreType.DMA,
    ],
    compiler_params=pltpu.CompilerParams(use_tc_tiling_on_sc=False),
)
def _bag_sum_sc(
    x_hbm, table_hbm, out_hbm, idx_v, g0, g1, g2, g3, acc_v, s0, s1, s2, s3
):
    wid = lax.axis_index("c") * _NS + lax.axis_index("s")
    bufs = ((g0, s0), (g1, s1), (g2, s2), (g3, s3))
    nbuf = len(bufs)
    pltpu.sync_copy(x_hbm.at[wid], idx_v)
    for b, (g, s) in enumerate(bufs):
        pltpu.async_copy(table_hbm.at[idx_v.at[b]], g, s)

    def reduce_chunk(g, chunk):
        for k in range(_CHUNK_BAGS):
            row = chunk * _CHUNK_BAGS + k
            for h in range(_DIM // _LANES):
                vals = [
                    g[k * _HIST + r, pl.ds(h * _LANES, _LANES)]
                    for r in range(_HIST)
                ]
                acc_v[row, pl.ds(h * _LANES, _LANES)] = _tree_add(vals)

    @pl.loop(0, _NCHUNK, step=nbuf)
    def _(j):
        for b, (g, s) in enumerate(bufs):
            pltpu.make_async_copy(table_hbm.at[idx_v.at[j + b]], g, s).wait()
            reduce_chunk(g, j + b)

            @pl.when(j + b + nbuf < _NCHUNK)
            def _():
                pltpu.async_copy(table_hbm.at[idx_v.at[j + b + nbuf]], g, s)

    pltpu.sync_copy(acc_v, out_hbm.at[pl.ds(wid * _BAGS_PER_W, _BAGS_PER_W)])


_TCW = 1024


_TPW = 2048
_TPB = 4 * _TPW
_TPMAIN = 122
_TPTAIL = 512
_TPROWS = (_TPMAIN + 1) * _TPW


def _tp_body(m0, m1, m2, m3, t0, t1, o_ref):
    pid = pl.program_id(0)

    @pl.when(pid < _TPMAIN)
    def _():
        o_ref[...] = jnp.concatenate(
            [jnp.transpose(r[...]) for r in (m0, m1, m2, m3)], axis=1
        )

    @pl.when(pid == _TPMAIN)
    def _():
        tail = jnp.concatenate(
            [
                jnp.transpose(t0[...]),
                jnp.transpose(t1[...]),
                jnp.zeros((_TPTAIL, 2 * _DIM), jnp.float32),
            ],
            axis=1,
        )
        o_ref[...] = jnp.concatenate(
            [tail, jnp.zeros((_TPW - _TPTAIL, 128), jnp.float32)], axis=0
        )


def _table_rowgroups_tc(tT):
    max_blk = (tT.shape[1] - _TPW) // _TPW
    in_specs = [
        pl.BlockSpec(
            (_DIM, _TPW), lambda i, k=k: (0, jnp.minimum(4 * i + k, max_blk))
        )
        for k in range(4)
    ]
    in_specs += [
        pl.BlockSpec((_DIM, _TPTAIL), lambda i, k=k: (0, 1952 + k))
        for k in range(2)
    ]
    return pl.pallas_call(
        _tp_body,
        grid=(_TPMAIN + 1,),
        in_specs=in_specs,
        out_specs=pl.BlockSpec((_TPW, 128), lambda i: (i, 0)),
        out_shape=jax.ShapeDtypeStruct((_TPROWS, 128), jnp.float32),
    )(tT, tT, tT, tT, tT, tT)


_BT = 512


def _mm_body(w_ref, e_ref, b_ref, o_ref):
    o_ref[...] = (
        lax.dot_general(
            w_ref[...],
            e_ref[...],
            (((1,), (1,)), ((), ())),
            preferred_element_type=jnp.float32,
        )
        + b_ref[...]
    )


def _linear_tc(embed_sum, wt, bcol):
    return pl.pallas_call(
        _mm_body,
        grid=(_BATCH // _BT,),
        in_specs=[
            pl.BlockSpec((_NCLASS, _DIM), lambda i: (0, 0)),
            pl.BlockSpec((_BT, _DIM), lambda i: (i, 0)),
            pl.BlockSpec((_NCLASS, 1), lambda i: (0, 0)),
        ],
        out_specs=pl.BlockSpec((_NCLASS, _BT), lambda i: (0, i)),
        out_shape=jax.ShapeDtypeStruct((_NCLASS, _BATCH), jnp.float32),
    )(wt, embed_sum, bcol)


def kernel(x, table, W, b):
    xi = x.astype(jnp.int32)
    b8 = xi % _TPB
    j_main = (xi - b8) + 4 * (b8 % _TPW) + b8 // _TPW
    t = xi - _TPMAIN * _TPB
    j_tail = _TPMAIN * _TPB + 4 * (t % _TPTAIL) + t // _TPTAIL
    xw = jnp.where(xi < _TPMAIN * _TPB, j_main, j_tail).reshape(
        _NW, _NCHUNK, _IDX_PER_CHUNK
    )
    tpack = _table_rowgroups_tc(jnp.transpose(table))
    tbl = jnp.reshape(tpack, (4 * _TPROWS, _DIM))
    embed_sum = _bag_sum_sc(xw, tbl)
    wt = W.astype(jnp.float32) * (1.0 / _HIST)
    bcol = b.reshape(_NCLASS, 1).astype(jnp.float32)
    out_t = _linear_tc(embed_sum, wt, bcol)
    return jnp.transpose(out_t)


# speedup vs baseline: 2.2600x; 1.0289x over previous
import functools

import jax
import jax.numpy as jnp
from jax import lax
from jax.experimental import pallas as pl
from jax.experimental.pallas import tpu as pltpu
from jax.experimental.pallas import tpu_sc as plsc

_BATCH = 16384
_HIST = 50
_DIM = 32
_NCLASS = 1000

_NC = 2
_NS = 16
_NW = _NC * _NS
_LANES = 16
_BAGS_PER_W = _BATCH // _NW
_CHUNK_BAGS = 2
_IDX_PER_CHUNK = _CHUNK_BAGS * _HIST
_NCHUNK = _BAGS_PER_W // _CHUNK_BAGS

_sc_mesh = plsc.VectorSubcoreMesh(core_axis_name="c", subcore_axis_name="s")


def _tree_add(vals):
    while len(vals) > 1:
        nxt = [a + b for a, b in zip(vals[0::2], vals[1::2])]
        if len(vals) % 2:
            nxt.append(vals[-1])
        vals = nxt
    return vals[0]


@functools.partial(
    pl.kernel,
    out_type=jax.ShapeDtypeStruct((_BATCH, _DIM), jnp.float32),
    mesh=_sc_mesh,
    scratch_types=[
        pltpu.VMEM((_NCHUNK, _IDX_PER_CHUNK), jnp.int32),
        pltpu.VMEM((_IDX_PER_CHUNK, _DIM), jnp.float32),
        pltpu.VMEM((_IDX_PER_CHUNK, _DIM), jnp.float32),
        pltpu.VMEM((_IDX_PER_CHUNK, _DIM), jnp.float32),
        pltpu.VMEM((_IDX_PER_CHUNK, _DIM), jnp.float32),
        pltpu.VMEM((_BAGS_PER_W, _DIM), jnp.float32),
        pltpu.SemaphoreType.DMA,
        pltpu.SemaphoreType.DMA,
        pltpu.SemaphoreType.DMA,
        pltpu.SemaphoreType.DMA,
    ],
    compiler_params=pltpu.CompilerParams(use_tc_tiling_on_sc=False),
)
def _bag_sum_sc(
    x_hbm, table_hbm, out_hbm, idx_v, g0, g1, g2, g3, acc_v, s0, s1, s2, s3
):
    wid = lax.axis_index("c") * _NS + lax.axis_index("s")
    bufs = ((g0, s0), (g1, s1), (g2, s2), (g3, s3))
    nbuf = len(bufs)
    pltpu.sync_copy(x_hbm.at[wid], idx_v)
    for b, (g, s) in enumerate(bufs):
        pltpu.async_copy(table_hbm.at[idx_v.at[b]], g, s)

    def reduce_chunk(g, chunk):
        for k in range(_CHUNK_BAGS):
            row = chunk * _CHUNK_BAGS + k
            for h in range(_DIM // _LANES):
                vals = [
                    g[k * _HIST + r, pl.ds(h * _LANES, _LANES)]
                    for r in range(_HIST)
                ]
                acc_v[row, pl.ds(h * _LANES, _LANES)] = _tree_add(vals)

    @pl.loop(0, _NCHUNK, step=nbuf)
    def _(j):
        for b, (g, s) in enumerate(bufs):
            pltpu.make_async_copy(table_hbm.at[idx_v.at[j + b]], g, s).wait()
            reduce_chunk(g, j + b)

            @pl.when(j + b + nbuf < _NCHUNK)
            def _():
                pltpu.async_copy(table_hbm.at[idx_v.at[j + b + nbuf]], g, s)

    pltpu.sync_copy(acc_v, out_hbm.at[pl.ds(wid * _BAGS_PER_W, _BAGS_PER_W)])


_TCW = 1024


_TPW = 4096
_TPB = 4 * _TPW
_TPMAIN = 61
_TPTAIL = 512
_TPROWS = (_TPMAIN + 1) * _TPW


def _tp_body(m0, m1, m2, m3, t0, t1, o_ref):
    pid = pl.program_id(0)

    @pl.when(pid < _TPMAIN)
    def _():
        o_ref[...] = jnp.concatenate(
            [jnp.transpose(r[...]) for r in (m0, m1, m2, m3)], axis=1
        )

    @pl.when(pid == _TPMAIN)
    def _():
        tail = jnp.concatenate(
            [
                jnp.transpose(t0[...]),
                jnp.transpose(t1[...]),
                jnp.zeros((_TPTAIL, 2 * _DIM), jnp.float32),
            ],
            axis=1,
        )
        o_ref[...] = jnp.concatenate(
            [tail, jnp.zeros((_TPW - _TPTAIL, 128), jnp.float32)], axis=0
        )


def _table_rowgroups_tc(tT):
    max_blk = (tT.shape[1] - _TPW) // _TPW
    in_specs = [
        pl.BlockSpec(
            (_DIM, _TPW), lambda i, k=k: (0, jnp.minimum(4 * i + k, max_blk))
        )
        for k in range(4)
    ]
    in_specs += [
        pl.BlockSpec((_DIM, _TPTAIL), lambda i, k=k: (0, 1952 + k))
        for k in range(2)
    ]
    return pl.pallas_call(
        _tp_body,
        grid=(_TPMAIN + 1,),
        in_specs=in_specs,
        out_specs=pl.BlockSpec((_TPW, 128), lambda i: (i, 0)),
        out_shape=jax.ShapeDtypeStruct((_TPROWS, 128), jnp.float32),
    )(tT, tT, tT, tT, tT, tT)


_BT = 512


def _mm_body(w_ref, e_ref, b_ref, o_ref):
    o_ref[...] = (
        lax.dot_general(
            w_ref[...],
            e_ref[...],
            (((1,), (1,)), ((), ())),
            preferred_element_type=jnp.float32,
        )
        + b_ref[...]
    )


def _linear_tc(embed_sum, wt, bcol):
    return pl.pallas_call(
        _mm_body,
        grid=(_BATCH // _BT,),
        in_specs=[
            pl.BlockSpec((_NCLASS, _DIM), lambda i: (0, 0)),
            pl.BlockSpec((_BT, _DIM), lambda i: (i, 0)),
            pl.BlockSpec((_NCLASS, 1), lambda i: (0, 0)),
        ],
        out_specs=pl.BlockSpec((_NCLASS, _BT), lambda i: (0, i)),
        out_shape=jax.ShapeDtypeStruct((_NCLASS, _BATCH), jnp.float32),
    )(wt, embed_sum, bcol)


def kernel(x, table, W, b):
    xi = x.astype(jnp.int32)
    b8 = xi % _TPB
    j_main = (xi - b8) + 4 * (b8 % _TPW) + b8 // _TPW
    t = xi - _TPMAIN * _TPB
    j_tail = _TPMAIN * _TPB + 4 * (t % _TPTAIL) + t // _TPTAIL
    xw = jnp.where(xi < _TPMAIN * _TPB, j_main, j_tail).reshape(
        _NW, _NCHUNK, _IDX_PER_CHUNK
    )
    tpack = _table_rowgroups_tc(jnp.transpose(table))
    tbl = jnp.reshape(tpack, (4 * _TPROWS, _DIM))
    embed_sum = _bag_sum_sc(xw, tbl)
    wt = W.astype(jnp.float32) * (1.0 / _HIST)
    bcol = b.reshape(_NCLASS, 1).astype(jnp.float32)
    out_t = _linear_tc(embed_sum, wt, bcol)
    return jnp.transpose(out_t)
